# Initial kernel scaffold; baseline (speedup 1.0000x reference)
import jax
import jax.numpy as jnp
from jax.experimental import pallas as pl


def kernel(preds, labels):
    raise NotImplementedError("write your pallas kernel here")


# speedup vs baseline: 33.7387x; 33.7387x over previous
import jax
import jax.numpy as jnp
from jax.experimental import pallas as pl
from jax.experimental.pallas import tpu as pltpu

NC = 9
_STD = 0.1
_AVG = 1.0


def _rw_kernel(labels_ref, preds_ref, out_ref, w_ref):
    nb = pl.program_id(1)

    @pl.when(nb == 0)
    def _():
        lab = labels_ref[0]
        npix = lab.shape[0] * lab.shape[1]
        hist = [jnp.sum((lab == c).astype(jnp.float32)) for c in range(NC)]
        h = [jnp.where(hc > 0.0, -jnp.log(hc / npix), 0.0) for hc in hist]
        cnt = sum(jnp.where(hc > 0.0, 1.0, 0.0) for hc in hist)
        mean = sum(h) / cnt
        var = sum(jnp.where(hc > 0.0, (hh - mean) ** 2, 0.0)
                  for hc, hh in zip(hist, h)) / cnt
        std = jnp.sqrt(var)
        for c in range(NC):
            w_ref[c] = jnp.where(h[c] != 0.0,
                                 (h[c] - mean) / std * _STD + _AVG, 1.0)

    for c in range(NC):
        out_ref[0, c] = preds_ref[0, c] * w_ref[c]


def kernel(preds, labels):
    B, C, H, W = preds.shape
    NB = 4
    HB = H // NB
    return pl.pallas_call(
        _rw_kernel,
        grid=(B, NB),
        in_specs=[
            pl.BlockSpec((1, H, W), lambda b, nb: (b, 0, 0)),
            pl.BlockSpec((1, C, HB, W), lambda b, nb: (b, 0, nb, 0)),
        ],
        out_specs=pl.BlockSpec((1, C, HB, W), lambda b, nb: (b, 0, nb, 0)),
        out_shape=jax.ShapeDtypeStruct((B, C, H, W), preds.dtype),
        scratch_shapes=[pltpu.SMEM((NC,), jnp.float32)],
    )(labels, preds)
